# baseline (speedup 1.0000x reference)
import functools

import jax
import jax.numpy as jnp
from jax import lax
from jax.experimental import pallas as pl
from jax.experimental.pallas import tpu as pltpu
from jax.experimental.pallas import tpu_sc as plsc

F32 = jnp.float32

_NC = 2
_NS = 16
_B = 128
_G = 1


def _sc_aggregate(F, NP, KB, rows_per_tile, with_deg):
    mesh = plsc.VectorSubcoreMesh(core_axis_name="c", subcore_axis_name="s")
    NG = KB // _G
    out_type = [jax.ShapeDtypeStruct((_NC, NP, F), F32)]
    scratch = [
        pltpu.VMEM_SHARED((NP, F), F32),
        pltpu.VMEM((KB, _B), jnp.int32),
        pltpu.VMEM((KB, _B), jnp.int32),
        [pltpu.VMEM((_B, F), F32)] * _G,
        [pltpu.VMEM((_B, F), F32)] * _G,
        pltpu.SemaphoreType.DMA,
        pltpu.SemaphoreType.DMA,
        pltpu.SemaphoreType.DMA,
        pltpu.SemaphoreType.DMA,
    ]
    if with_deg:
        out_type.append(jax.ShapeDtypeStruct((_NC * _NS, NP), F32))
        scratch.append(pltpu.VMEM((NP,), F32))

    @functools.partial(
        pl.kernel,
        out_type=out_type,
        mesh=mesh,
        scratch_types=scratch,
        compiler_params=pltpu.CompilerParams(use_tc_tiling_on_sc=False,
                                             needs_layout_passes=False),
    )
    def sc(y_hbm, srcb, dstb, zer, zdeg, out, *rest):
        if with_deg:
            (deg_out, agg_sh, src_v, dst_v, bufs_p, bufs_q,
             gsem_p, gsem_q, ssem_p, ssem_q, deg_loc) = rest
        else:
            (agg_sh, src_v, dst_v, bufs_p, bufs_q,
             gsem_p, gsem_q, ssem_p, ssem_q) = rest
        cid = lax.axis_index("c")
        sid = lax.axis_index("s")
        wid = cid * _NS + sid
        r0 = sid * rows_per_tile
        pltpu.sync_copy(zer.at[pl.ds(r0, rows_per_tile)],
                        agg_sh.at[pl.ds(r0, rows_per_tile)])
        if with_deg:
            pltpu.sync_copy(zdeg, deg_loc)
        pltpu.sync_copy(srcb.at[wid], src_v)
        pltpu.sync_copy(dstb.at[wid], dst_v)
        plsc.subcore_barrier()

        ones16 = jnp.ones((16,), F32)

        def g_start(q, bufs, sem):
            for k in range(_G):
                pltpu.async_copy(y_hbm.at[src_v.at[_G * q + k]], bufs[k], sem)

        def g_wait(q, bufs, sem):
            for k in range(_G):
                pltpu.make_async_copy(
                    y_hbm.at[src_v.at[_G * q + k]], bufs[k], sem).wait()

        def s_start(q, bufs, sem):
            for k in range(_G):
                pltpu.async_copy(bufs[k], agg_sh.at[dst_v.at[_G * q + k]],
                                 sem, add=True)

        def s_wait(q, bufs, sem):
            for k in range(_G):
                pltpu.make_async_copy(
                    bufs[k], agg_sh.at[dst_v.at[_G * q + k]], sem).wait()

        def deg_acc(q):
            if with_deg:
                for k in range(_G):
                    for c in range(_B // 16):
                        idx = dst_v[_G * q + k, pl.ds(16 * c, 16)]
                        plsc.addupdate_scatter(deg_loc, [idx], ones16)

        g_start(0, bufs_p, gsem_p)
        g_wait(0, bufs_p, gsem_p)
        s_start(0, bufs_p, ssem_p)
        g_start(1, bufs_q, gsem_q)
        deg_acc(0)
        g_wait(1, bufs_q, gsem_q)
        s_wait(0, bufs_p, ssem_p)
        s_start(1, bufs_q, ssem_q)
        g_start(2, bufs_p, gsem_p)
        deg_acc(1)

        def body(u, carry):
            qa = 2 * u
            qb = qa + 1
            g_wait(qa, bufs_p, gsem_p)
            s_wait(qa - 1, bufs_q, ssem_q)
            s_start(qa, bufs_p, ssem_p)
            g_start(qb, bufs_q, gsem_q)
            deg_acc(qa)
            g_wait(qb, bufs_q, gsem_q)
            s_wait(qa, bufs_p, ssem_p)
            s_start(qb, bufs_q, ssem_q)
            g_start(qa + 2, bufs_p, gsem_p)
            deg_acc(qb)
            return carry

        lax.fori_loop(1, NG // 2 - 1, body, 0)
        qa = NG - 2
        g_wait(qa, bufs_p, gsem_p)
        s_wait(qa - 1, bufs_q, ssem_q)
        s_start(qa, bufs_p, ssem_p)
        g_start(qa + 1, bufs_q, gsem_q)
        deg_acc(qa)
        g_wait(qa + 1, bufs_q, gsem_q)
        s_wait(qa, bufs_p, ssem_p)
        s_start(qa + 1, bufs_q, ssem_q)
        deg_acc(qa + 1)
        s_wait(qa + 1, bufs_q, ssem_q)
        plsc.subcore_barrier()
        pltpu.sync_copy(agg_sh.at[pl.ds(r0, rows_per_tile)],
                        out.at[cid, pl.ds(r0, rows_per_tile)])
        if with_deg:
            pltpu.sync_copy(deg_loc, deg_out.at[wid])

    return sc


def _tc_a(x_ref, wl_ref, wr_ref, y_ref, z_ref):
    xb = x_ref[...]
    y_ref[...] = jnp.dot(xb, wl_ref[...], preferred_element_type=F32)
    z_ref[...] = jnp.dot(xb, wr_ref[...], preferred_element_type=F32)


def _deg_col(deg_ref):
    return jnp.maximum(jnp.sum(deg_ref[...], axis=1, keepdims=True), 1.0)


def _tc_b(agg_ref, deg_ref, z_ref, b1_ref, w2l_ref, w2r_ref, y2_ref, z2_ref):
    a = agg_ref[0] + agg_ref[1]
    deg = _deg_col(deg_ref)
    h = jnp.maximum(a / deg + b1_ref[...] + z_ref[...], 0.0)
    y2_ref[...] = jnp.dot(h, w2l_ref[...], preferred_element_type=F32)
    z2_ref[...] = jnp.dot(h, w2r_ref[...], preferred_element_type=F32)


def _tc_c(agg_ref, deg_ref, z_ref, b2_ref, wl_ref, bl_ref, h_ref, out_ref):
    a = agg_ref[0] + agg_ref[1]
    deg = _deg_col(deg_ref)
    h = jnp.maximum(a / deg + b2_ref[...] + z_ref[...], 0.0)
    h_ref[...] = h
    out_ref[...] = jnp.dot(h, wl_ref[...], preferred_element_type=F32) + bl_ref[...]


def kernel(x, edge_index, W1l, b1, W1r, W2l, b2, W2r, Wlin, blin):
    N, D = x.shape
    H = W1l.shape[0]
    E = edge_index.shape[1]

    NW = _NC * _NS
    KB = -(-E // (NW * _B * 2 * _G)) * 2 * _G
    EP = NW * KB * _B
    rows_per_tile = -(-(N + 1) // _NS // 8) * 8
    NP = _NS * rows_per_tile

    src = edge_index[0]
    dst = edge_index[1]
    pad = EP - E
    srcb = jnp.concatenate([src, jnp.zeros((pad,), jnp.int32)]).reshape(NW, KB, _B)
    dstb = jnp.concatenate([dst, jnp.full((pad,), N, jnp.int32)]).reshape(NW, KB, _B)

    zer = jnp.zeros((NP, H), F32)
    zdeg = jnp.zeros((NP,), F32)

    BN = 2000
    NB = N // BN

    y1, z1 = pl.pallas_call(
        _tc_a,
        grid=(NB,),
        in_specs=[
            pl.BlockSpec((BN, D), lambda i: (i, 0)),
            pl.BlockSpec((D, H), lambda i: (0, 0)),
            pl.BlockSpec((D, H), lambda i: (0, 0)),
        ],
        out_specs=[
            pl.BlockSpec((BN, H), lambda i: (i, 0)),
            pl.BlockSpec((BN, H), lambda i: (i, 0)),
        ],
        out_shape=[
            jax.ShapeDtypeStruct((N, H), F32),
            jax.ShapeDtypeStruct((N, H), F32),
        ],
    )(x, W1l.T, W1r.T)

    agg1, deg1 = _sc_aggregate(H, NP, KB, rows_per_tile, True)(
        y1, srcb, dstb, zer, zdeg)
    deg1 = deg1.T

    y2, z2 = pl.pallas_call(
        _tc_b,
        grid=(NB,),
        in_specs=[
            pl.BlockSpec((_NC, BN, H), lambda i: (0, i, 0)),
            pl.BlockSpec((BN, NW), lambda i: (i, 0)),
            pl.BlockSpec((BN, H), lambda i: (i, 0)),
            pl.BlockSpec((1, H), lambda i: (0, 0)),
            pl.BlockSpec((H, H), lambda i: (0, 0)),
            pl.BlockSpec((H, H), lambda i: (0, 0)),
        ],
        out_specs=[
            pl.BlockSpec((BN, H), lambda i: (i, 0)),
            pl.BlockSpec((BN, H), lambda i: (i, 0)),
        ],
        out_shape=[
            jax.ShapeDtypeStruct((N, H), F32),
            jax.ShapeDtypeStruct((N, H), F32),
        ],
    )(agg1, deg1, z1, b1.reshape(1, H), W2l.T, W2r.T)

    (agg2,) = _sc_aggregate(H, NP, KB, rows_per_tile, False)(
        y2, srcb, dstb, zer, zdeg)

    h2, out2d = pl.pallas_call(
        _tc_c,
        grid=(NB,),
        in_specs=[
            pl.BlockSpec((_NC, BN, H), lambda i: (0, i, 0)),
            pl.BlockSpec((BN, NW), lambda i: (i, 0)),
            pl.BlockSpec((BN, H), lambda i: (i, 0)),
            pl.BlockSpec((1, H), lambda i: (0, 0)),
            pl.BlockSpec((H, 1), lambda i: (0, 0)),
            pl.BlockSpec((1, 1), lambda i: (0, 0)),
        ],
        out_specs=[
            pl.BlockSpec((BN, H), lambda i: (i, 0)),
            pl.BlockSpec((BN, 1), lambda i: (i, 0)),
        ],
        out_shape=[
            jax.ShapeDtypeStruct((N, H), F32),
            jax.ShapeDtypeStruct((N, 1), F32),
        ],
    )(agg2, deg1, z2, b2.reshape(1, H), Wlin.T, blin.reshape(1, 1))

    return (out2d[:, 0], h2)


# speedup vs baseline: 1.2036x; 1.2036x over previous
import functools

import jax
import jax.numpy as jnp
from jax import lax
from jax.experimental import pallas as pl
from jax.experimental.pallas import tpu as pltpu
from jax.experimental.pallas import tpu_sc as plsc

F32 = jnp.float32

_NC = 2
_NS = 16
_B = 128


def _sc_aggregate(F, NP, KB, rows_per_tile):
    mesh = plsc.VectorSubcoreMesh(core_axis_name="c", subcore_axis_name="s")

    @functools.partial(
        pl.kernel,
        out_type=jax.ShapeDtypeStruct((_NC, NP, F), F32),
        mesh=mesh,
        scratch_types=[
            pltpu.VMEM_SHARED((NP, F), F32),
            pltpu.VMEM((KB, _B), jnp.int32),
            pltpu.VMEM((KB, _B), jnp.int32),
            pltpu.VMEM((_B, F), F32),
            pltpu.VMEM((_B, F), F32),
            pltpu.SemaphoreType.DMA,
            pltpu.SemaphoreType.DMA,
        ],
        compiler_params=pltpu.CompilerParams(use_tc_tiling_on_sc=False),
    )
    def sc(y_hbm, srcb, dstb, zer, out, agg_sh, src_v, dst_v,
           rows_a, rows_b, sem_a, sem_b):
        cid = lax.axis_index("c")
        sid = lax.axis_index("s")
        wid = cid * _NS + sid
        r0 = sid * rows_per_tile
        pltpu.sync_copy(zer.at[pl.ds(r0, rows_per_tile)],
                        agg_sh.at[pl.ds(r0, rows_per_tile)])
        pltpu.sync_copy(srcb.at[wid], src_v)
        pltpu.sync_copy(dstb.at[wid], dst_v)
        plsc.subcore_barrier()

        def gather(j, buf, sem):
            return pltpu.make_async_copy(y_hbm.at[src_v.at[j]], buf, sem)

        def scatter(j, buf):
            pltpu.sync_copy(buf, agg_sh.at[dst_v.at[j]], add=True)

        assert KB % 2 == 1
        gather(0, rows_a, sem_a).start()

        def body(t, carry):
            j = 2 * t
            gather(j, rows_a, sem_a).wait()
            gather(j + 1, rows_b, sem_b).start()
            scatter(j, rows_a)
            gather(j + 1, rows_b, sem_b).wait()
            gather(j + 2, rows_a, sem_a).start()
            scatter(j + 1, rows_b)
            return carry

        lax.fori_loop(0, (KB - 1) // 2, body, 0)
        gather(KB - 1, rows_a, sem_a).wait()
        scatter(KB - 1, rows_a)
        plsc.subcore_barrier()
        pltpu.sync_copy(agg_sh.at[pl.ds(r0, rows_per_tile)],
                        out.at[cid, pl.ds(r0, rows_per_tile)])

    return sc


def _tc_a(x_ref, wae_ref, wrt_ref, yext_ref, z_ref):
    xb = x_ref[...]
    y = jnp.dot(xb, wae_ref[...], preferred_element_type=F32)
    cols = lax.broadcasted_iota(jnp.int32, y.shape, 1)
    yext_ref[...] = y + (cols == 64).astype(F32)
    z_ref[...] = jnp.dot(xb, wrt_ref[...], preferred_element_type=F32)


def _tc_b(agg_ref, z_ref, b1_ref, w2l_ref, w2r_ref, y2_ref, z2_ref):
    a = agg_ref[0] + agg_ref[1]
    deg = jnp.maximum(a[:, 64:65], 1.0)
    h = jnp.maximum(a[:, :64] / deg + b1_ref[...] + z_ref[...], 0.0)
    y2_ref[...] = jnp.dot(h, w2l_ref[...], preferred_element_type=F32)
    z2_ref[...] = jnp.dot(h, w2r_ref[...], preferred_element_type=F32)


def _tc_c(agg1_ref, agg2_ref, z_ref, b2_ref, wl_ref, bl_ref, h_ref, out_ref):
    a1 = agg1_ref[0] + agg1_ref[1]
    a2 = agg2_ref[0] + agg2_ref[1]
    deg = jnp.maximum(a1[:, 64:65], 1.0)
    h = jnp.maximum(a2 / deg + b2_ref[...] + z_ref[...], 0.0)
    h_ref[...] = h
    out_ref[...] = jnp.dot(h, wl_ref[...], preferred_element_type=F32) + bl_ref[...]


def kernel(x, edge_index, W1l, b1, W1r, W2l, b2, W2r, Wlin, blin):
    N, D = x.shape
    H = W1l.shape[0]
    E = edge_index.shape[1]
    FE = 80

    NW = _NC * _NS
    KB = -(-E // (NW * _B))
    EP = NW * KB * _B
    rows_per_tile = -(-(N + 1) // _NS // 8) * 8
    NP = _NS * rows_per_tile

    src = edge_index[0]
    dst = edge_index[1]
    pad = EP - E
    srcb = jnp.concatenate([src, jnp.zeros((pad,), jnp.int32)]).reshape(NW, KB, _B)
    dstb = jnp.concatenate([dst, jnp.full((pad,), N, jnp.int32)]).reshape(NW, KB, _B)

    zer80 = jnp.zeros((NP, FE), F32)
    zer64 = jnp.zeros((NP, H), F32)
    wae = jnp.concatenate([W1l.T, jnp.zeros((D, FE - H), F32)], axis=1)

    BN = 2000
    NB = N // BN

    yext, z1 = pl.pallas_call(
        _tc_a,
        grid=(NB,),
        in_specs=[
            pl.BlockSpec((BN, D), lambda i: (i, 0)),
            pl.BlockSpec((D, FE), lambda i: (0, 0)),
            pl.BlockSpec((D, H), lambda i: (0, 0)),
        ],
        out_specs=[
            pl.BlockSpec((BN, FE), lambda i: (i, 0)),
            pl.BlockSpec((BN, H), lambda i: (i, 0)),
        ],
        out_shape=[
            jax.ShapeDtypeStruct((N, FE), F32),
            jax.ShapeDtypeStruct((N, H), F32),
        ],
    )(x, wae, W1r.T)

    agg1 = _sc_aggregate(FE, NP, KB, rows_per_tile)(yext, srcb, dstb, zer80)

    y2, z2 = pl.pallas_call(
        _tc_b,
        grid=(NB,),
        in_specs=[
            pl.BlockSpec((_NC, BN, FE), lambda i: (0, i, 0)),
            pl.BlockSpec((BN, H), lambda i: (i, 0)),
            pl.BlockSpec((1, H), lambda i: (0, 0)),
            pl.BlockSpec((H, H), lambda i: (0, 0)),
            pl.BlockSpec((H, H), lambda i: (0, 0)),
        ],
        out_specs=[
            pl.BlockSpec((BN, H), lambda i: (i, 0)),
            pl.BlockSpec((BN, H), lambda i: (i, 0)),
        ],
        out_shape=[
            jax.ShapeDtypeStruct((N, H), F32),
            jax.ShapeDtypeStruct((N, H), F32),
        ],
    )(agg1, z1, b1.reshape(1, H), W2l.T, W2r.T)

    agg2 = _sc_aggregate(H, NP, KB, rows_per_tile)(y2, srcb, dstb, zer64)

    h2, out2d = pl.pallas_call(
        _tc_c,
        grid=(NB,),
        in_specs=[
            pl.BlockSpec((_NC, BN, FE), lambda i: (0, i, 0)),
            pl.BlockSpec((_NC, BN, H), lambda i: (0, i, 0)),
            pl.BlockSpec((BN, H), lambda i: (i, 0)),
            pl.BlockSpec((1, H), lambda i: (0, 0)),
            pl.BlockSpec((H, 1), lambda i: (0, 0)),
            pl.BlockSpec((1, 1), lambda i: (0, 0)),
        ],
        out_specs=[
            pl.BlockSpec((BN, H), lambda i: (i, 0)),
            pl.BlockSpec((BN, 1), lambda i: (i, 0)),
        ],
        out_shape=[
            jax.ShapeDtypeStruct((N, H), F32),
            jax.ShapeDtypeStruct((N, 1), F32),
        ],
    )(agg1, agg2, z2, b2.reshape(1, H), Wlin.T, blin.reshape(1, 1))

    return (out2d[:, 0], h2)


# speedup vs baseline: 1.2266x; 1.0191x over previous
import functools

import jax
import jax.numpy as jnp
from jax import lax
from jax.experimental import pallas as pl
from jax.experimental.pallas import tpu as pltpu
from jax.experimental.pallas import tpu_sc as plsc

F32 = jnp.float32

_NC = 1
_NS = 16
_B = 128


def _sc_aggregate(F, NP, KB, rows_per_tile):
    mesh = plsc.VectorSubcoreMesh(core_axis_name="c", subcore_axis_name="s",
                                  num_cores=_NC)

    @functools.partial(
        pl.kernel,
        out_type=jax.ShapeDtypeStruct((_NC, NP, F), F32),
        mesh=mesh,
        scratch_types=[
            pltpu.VMEM_SHARED((NP, F), F32),
            pltpu.VMEM((KB, _B), jnp.int32),
            pltpu.VMEM((KB, _B), jnp.int32),
            pltpu.VMEM((_B, F), F32),
            pltpu.VMEM((_B, F), F32),
            pltpu.SemaphoreType.DMA,
            pltpu.SemaphoreType.DMA,
        ],
        compiler_params=pltpu.CompilerParams(use_tc_tiling_on_sc=False),
    )
    def sc(y_hbm, srcb, dstb, zer, out, agg_sh, src_v, dst_v,
           rows_a, rows_b, sem_a, sem_b):
        cid = lax.axis_index("c")
        sid = lax.axis_index("s")
        wid = cid * _NS + sid
        r0 = sid * rows_per_tile
        pltpu.sync_copy(zer.at[pl.ds(r0, rows_per_tile)],
                        agg_sh.at[pl.ds(r0, rows_per_tile)])
        pltpu.sync_copy(srcb.at[wid], src_v)
        pltpu.sync_copy(dstb.at[wid], dst_v)
        plsc.subcore_barrier()

        def gather(j, buf, sem):
            return pltpu.make_async_copy(y_hbm.at[src_v.at[j]], buf, sem)

        def scatter(j, buf):
            pltpu.sync_copy(buf, agg_sh.at[dst_v.at[j]], add=True)

        assert KB % 2 == 1
        gather(0, rows_a, sem_a).start()

        def body(t, carry):
            j = 2 * t
            gather(j, rows_a, sem_a).wait()
            gather(j + 1, rows_b, sem_b).start()
            scatter(j, rows_a)
            gather(j + 1, rows_b, sem_b).wait()
            gather(j + 2, rows_a, sem_a).start()
            scatter(j + 1, rows_b)
            return carry

        lax.fori_loop(0, (KB - 1) // 2, body, 0)
        gather(KB - 1, rows_a, sem_a).wait()
        scatter(KB - 1, rows_a)
        plsc.subcore_barrier()
        pltpu.sync_copy(agg_sh.at[pl.ds(r0, rows_per_tile)],
                        out.at[cid, pl.ds(r0, rows_per_tile)])

    return sc


def _tc_a(x_ref, wae_ref, wrt_ref, yext_ref, z_ref):
    xb = x_ref[...]
    y = jnp.dot(xb, wae_ref[...], preferred_element_type=F32)
    cols = lax.broadcasted_iota(jnp.int32, y.shape, 1)
    yext_ref[...] = y + (cols == 64).astype(F32)
    z_ref[...] = jnp.dot(xb, wrt_ref[...], preferred_element_type=F32)


def _tc_b(agg_ref, z_ref, b1_ref, w2l_ref, w2r_ref, y2_ref, z2_ref):
    a = jnp.sum(agg_ref[...], axis=0)
    deg = jnp.maximum(a[:, 64:65], 1.0)
    h = jnp.maximum(a[:, :64] / deg + b1_ref[...] + z_ref[...], 0.0)
    y2_ref[...] = jnp.dot(h, w2l_ref[...], preferred_element_type=F32)
    z2_ref[...] = jnp.dot(h, w2r_ref[...], preferred_element_type=F32)


def _tc_c(agg1_ref, agg2_ref, z_ref, b2_ref, wl_ref, bl_ref, h_ref, out_ref):
    a1 = jnp.sum(agg1_ref[...], axis=0)
    a2 = jnp.sum(agg2_ref[...], axis=0)
    deg = jnp.maximum(a1[:, 64:65], 1.0)
    h = jnp.maximum(a2 / deg + b2_ref[...] + z_ref[...], 0.0)
    h_ref[...] = h
    out_ref[...] = jnp.dot(h, wl_ref[...], preferred_element_type=F32) + bl_ref[...]


def kernel(x, edge_index, W1l, b1, W1r, W2l, b2, W2r, Wlin, blin):
    N, D = x.shape
    H = W1l.shape[0]
    E = edge_index.shape[1]
    FE = 80

    NW = _NC * _NS
    KB = -(-E // (NW * _B))
    EP = NW * KB * _B
    rows_per_tile = -(-(N + 1) // _NS // 8) * 8
    NP = _NS * rows_per_tile

    src = edge_index[0]
    dst = edge_index[1]
    pad = EP - E
    srcb = jnp.concatenate([src, jnp.zeros((pad,), jnp.int32)]).reshape(NW, KB, _B)
    dstb = jnp.concatenate([dst, jnp.full((pad,), N, jnp.int32)]).reshape(NW, KB, _B)

    zer80 = jnp.zeros((NP, FE), F32)
    zer64 = jnp.zeros((NP, H), F32)
    wae = jnp.concatenate([W1l.T, jnp.zeros((D, FE - H), F32)], axis=1)

    BN = 2000
    NB = N // BN

    yext, z1 = pl.pallas_call(
        _tc_a,
        grid=(NB,),
        in_specs=[
            pl.BlockSpec((BN, D), lambda i: (i, 0)),
            pl.BlockSpec((D, FE), lambda i: (0, 0)),
            pl.BlockSpec((D, H), lambda i: (0, 0)),
        ],
        out_specs=[
            pl.BlockSpec((BN, FE), lambda i: (i, 0)),
            pl.BlockSpec((BN, H), lambda i: (i, 0)),
        ],
        out_shape=[
            jax.ShapeDtypeStruct((N, FE), F32),
            jax.ShapeDtypeStruct((N, H), F32),
        ],
    )(x, wae, W1r.T)

    agg1 = _sc_aggregate(FE, NP, KB, rows_per_tile)(yext, srcb, dstb, zer80)

    y2, z2 = pl.pallas_call(
        _tc_b,
        grid=(NB,),
        in_specs=[
            pl.BlockSpec((_NC, BN, FE), lambda i: (0, i, 0)),
            pl.BlockSpec((BN, H), lambda i: (i, 0)),
            pl.BlockSpec((1, H), lambda i: (0, 0)),
            pl.BlockSpec((H, H), lambda i: (0, 0)),
            pl.BlockSpec((H, H), lambda i: (0, 0)),
        ],
        out_specs=[
            pl.BlockSpec((BN, H), lambda i: (i, 0)),
            pl.BlockSpec((BN, H), lambda i: (i, 0)),
        ],
        out_shape=[
            jax.ShapeDtypeStruct((N, H), F32),
            jax.ShapeDtypeStruct((N, H), F32),
        ],
    )(agg1, z1, b1.reshape(1, H), W2l.T, W2r.T)

    agg2 = _sc_aggregate(H, NP, KB, rows_per_tile)(y2, srcb, dstb, zer64)

    h2, out2d = pl.pallas_call(
        _tc_c,
        grid=(NB,),
        in_specs=[
            pl.BlockSpec((_NC, BN, FE), lambda i: (0, i, 0)),
            pl.BlockSpec((_NC, BN, H), lambda i: (0, i, 0)),
            pl.BlockSpec((BN, H), lambda i: (i, 0)),
            pl.BlockSpec((1, H), lambda i: (0, 0)),
            pl.BlockSpec((H, 1), lambda i: (0, 0)),
            pl.BlockSpec((1, 1), lambda i: (0, 0)),
        ],
        out_specs=[
            pl.BlockSpec((BN, H), lambda i: (i, 0)),
            pl.BlockSpec((BN, 1), lambda i: (i, 0)),
        ],
        out_shape=[
            jax.ShapeDtypeStruct((N, H), F32),
            jax.ShapeDtypeStruct((N, 1), F32),
        ],
    )(agg1, agg2, z2, b2.reshape(1, H), Wlin.T, blin.reshape(1, 1))

    return (out2d[:, 0], h2)
